# baseline (speedup 1.0000x reference)
import jax
import jax.numpy as jnp
from jax.experimental import pallas as pl

_BP = 512


def _add_kernel(x_ref, emb_ref, o_ref):
    o_ref[...] = x_ref[...] + emb_ref[...][None, :, :]


def kernel(x, emb):
    B, P, D = x.shape
    grid = (P // _BP, B)
    return pl.pallas_call(
        _add_kernel,
        grid=grid,
        in_specs=[
            pl.BlockSpec((1, _BP, D), lambda i, b: (b, i, 0)),
            pl.BlockSpec((_BP, D), lambda i, b: (i, 0)),
        ],
        out_specs=pl.BlockSpec((1, _BP, D), lambda i, b: (b, i, 0)),
        out_shape=jax.ShapeDtypeStruct((B, P, D), x.dtype),
    )(x, emb)


# speedup vs baseline: 1.2559x; 1.2559x over previous
import jax
import jax.numpy as jnp
from jax.experimental import pallas as pl

_BP = 1024


def _add_kernel(x_ref, emb_ref, o_ref):
    o_ref[...] = x_ref[...] + emb_ref[...][None, :, :]


def kernel(x, emb):
    B, P, D = x.shape
    grid = (P // _BP,)
    return pl.pallas_call(
        _add_kernel,
        grid=grid,
        in_specs=[
            pl.BlockSpec((B, _BP, D), lambda i: (0, i, 0)),
            pl.BlockSpec((_BP, D), lambda i: (i, 0)),
        ],
        out_specs=pl.BlockSpec((B, _BP, D), lambda i: (0, i, 0)),
        out_shape=jax.ShapeDtypeStruct((B, P, D), x.dtype),
    )(x, emb)
